# Initial kernel scaffold; baseline (speedup 1.0000x reference)
import functools

import jax
import jax.numpy as jnp
from jax import lax
from jax.experimental import pallas as pl
from jax.experimental.pallas import tpu as pltpu
from jax.experimental.pallas import tpu_sc as plsc

NUM_ROWS = 16384
NUM_FEATS = 26
DIM = 32

_NC = 2
_NS = 16
_NW = _NC * _NS

_B = NUM_ROWS * NUM_FEATS
_PER_W = _B // _NW
_CHUNK = 128
_NCHUNK = _PER_W // _CHUNK
_NBUF = 2


def _gather_body(idx_hbm, table_hbm, out_hbm, idx_v, buf0, buf1, sem0, sem1):
    wid = lax.axis_index("s") * _NC + lax.axis_index("c")
    pltpu.sync_copy(idx_hbm.at[wid], idx_v)
    bufs = (buf0, buf1)
    sems = (sem0, sem1)

    def start(j, b):
        pltpu.async_copy(table_hbm.at[idx_v.at[j]], bufs[b], sems[b])

    def finish(j, b):
        pltpu.make_async_copy(table_hbm.at[idx_v.at[j]], bufs[b], sems[b]).wait()
        pltpu.sync_copy(bufs[b], out_hbm.at[wid, j])

    for b in range(_NBUF):
        start(b, b)

    @pl.loop(0, _NCHUNK - _NBUF, step=_NBUF)
    def _(j0):
        for b in range(_NBUF):
            j = j0 + b
            finish(j, b)
            start(j + _NBUF, b)

    for b in range(_NBUF):
        finish(_NCHUNK - _NBUF + b, b)


@functools.partial(
    pl.kernel,
    out_type=jax.ShapeDtypeStruct((_NW, _NCHUNK, _CHUNK, DIM), jnp.float32),
    mesh=plsc.VectorSubcoreMesh(core_axis_name="c", subcore_axis_name="s"),
    scratch_types=[
        pltpu.VMEM((_NCHUNK, _CHUNK), jnp.int32),
        pltpu.VMEM((_CHUNK, DIM), jnp.float32),
        pltpu.VMEM((_CHUNK, DIM), jnp.float32),
        pltpu.SemaphoreType.DMA,
        pltpu.SemaphoreType.DMA,
    ],
)
def _gather(idx_hbm, table_hbm, out_hbm, idx_v, buf0, buf1, sem0, sem1):
    _gather_body(idx_hbm, table_hbm, out_hbm, idx_v, buf0, buf1, sem0, sem1)


def kernel(indices, table):
    idx = indices.reshape(_NW, _NCHUNK, _CHUNK).astype(jnp.int32)
    out = _gather(idx, table)
    return out.reshape(NUM_ROWS, NUM_FEATS, DIM)


# speedup vs baseline: 1.5238x; 1.5238x over previous
import functools

import jax
import jax.numpy as jnp
from jax import lax
from jax.experimental import pallas as pl
from jax.experimental.pallas import tpu as pltpu
from jax.experimental.pallas import tpu_sc as plsc

NUM_ROWS = 16384
NUM_FEATS = 26
DIM = 32

_NC = 2
_NS = 16
_NW = _NC * _NS

_B = NUM_ROWS * NUM_FEATS
_PER_W = _B // _NW
_CHUNK = 128
_NCHUNK = _PER_W // _CHUNK
_NBUF = 2


def _gather_body(idx_hbm, table_hbm, out_hbm, idx_v, buf0, buf1, sem0, sem1):
    wid = lax.axis_index("s") * _NC + lax.axis_index("c")
    pltpu.sync_copy(idx_hbm.at[wid], idx_v)
    bufs = (buf0, buf1)
    sems = (sem0, sem1)

    def start(j, b):
        pltpu.async_copy(table_hbm.at[idx_v.at[j]], bufs[b], sems[b])

    def finish(j, b):
        pltpu.make_async_copy(table_hbm.at[idx_v.at[j]], bufs[b], sems[b]).wait()
        pltpu.sync_copy(bufs[b], out_hbm.at[wid, j])

    for b in range(_NBUF):
        start(b, b)

    @pl.loop(0, _NCHUNK - _NBUF, step=_NBUF)
    def _(j0):
        for b in range(_NBUF):
            j = j0 + b
            finish(j, b)
            start(j + _NBUF, b)

    for b in range(_NBUF):
        finish(_NCHUNK - _NBUF + b, b)


@functools.partial(
    pl.kernel,
    out_type=jax.ShapeDtypeStruct((_NW, _NCHUNK, _CHUNK, DIM), jnp.float32),
    mesh=plsc.VectorSubcoreMesh(core_axis_name="c", subcore_axis_name="s"),
    scratch_types=[
        pltpu.VMEM((_NCHUNK, _CHUNK), jnp.int32),
        pltpu.VMEM((_CHUNK, DIM), jnp.float32),
        pltpu.VMEM((_CHUNK, DIM), jnp.float32),
        pltpu.SemaphoreType.DMA,
        pltpu.SemaphoreType.DMA,
    ],
    compiler_params=pltpu.CompilerParams(use_tc_tiling_on_sc=False),
)
def _gather(idx_hbm, table_hbm, out_hbm, idx_v, buf0, buf1, sem0, sem1):
    _gather_body(idx_hbm, table_hbm, out_hbm, idx_v, buf0, buf1, sem0, sem1)


def kernel(indices, table):
    idx = indices.reshape(_NW, _NCHUNK, _CHUNK).astype(jnp.int32)
    out = _gather(idx, table)
    return out.reshape(NUM_ROWS, NUM_FEATS, DIM)


# speedup vs baseline: 1.5740x; 1.0329x over previous
import functools

import jax
import jax.numpy as jnp
from jax import lax
from jax.experimental import pallas as pl
from jax.experimental.pallas import tpu as pltpu
from jax.experimental.pallas import tpu_sc as plsc

NUM_ROWS = 16384
NUM_FEATS = 26
DIM = 32

_NC = 2
_NS = 16
_NW = _NC * _NS

_B = NUM_ROWS * NUM_FEATS
_PER_W = _B // _NW
_CHUNK = 512
_NCHUNK = _PER_W // _CHUNK
_NBUF = 2


def _gather_body(idx_hbm, table_hbm, out_hbm, idx_v, buf0, buf1, sem0, sem1):
    wid = lax.axis_index("s") * _NC + lax.axis_index("c")
    pltpu.sync_copy(idx_hbm.at[wid], idx_v)
    bufs = (buf0, buf1)
    sems = (sem0, sem1)

    def start(j, b):
        pltpu.async_copy(table_hbm.at[idx_v.at[j]], bufs[b], sems[b])

    def finish(j, b):
        pltpu.make_async_copy(table_hbm.at[idx_v.at[j]], bufs[b], sems[b]).wait()
        pltpu.sync_copy(bufs[b], out_hbm.at[wid, j])

    for b in range(_NBUF):
        start(b, b)

    @pl.loop(0, _NCHUNK - _NBUF, step=_NBUF)
    def _(j0):
        for b in range(_NBUF):
            j = j0 + b
            finish(j, b)
            start(j + _NBUF, b)

    for b in range(_NBUF):
        finish(_NCHUNK - _NBUF + b, b)


@functools.partial(
    pl.kernel,
    out_type=jax.ShapeDtypeStruct((_NW, _NCHUNK, _CHUNK, DIM), jnp.float32),
    mesh=plsc.VectorSubcoreMesh(core_axis_name="c", subcore_axis_name="s"),
    scratch_types=[
        pltpu.VMEM((_NCHUNK, _CHUNK), jnp.int32),
        pltpu.VMEM((_CHUNK, DIM), jnp.float32),
        pltpu.VMEM((_CHUNK, DIM), jnp.float32),
        pltpu.SemaphoreType.DMA,
        pltpu.SemaphoreType.DMA,
    ],
    compiler_params=pltpu.CompilerParams(use_tc_tiling_on_sc=False),
)
def _gather(idx_hbm, table_hbm, out_hbm, idx_v, buf0, buf1, sem0, sem1):
    _gather_body(idx_hbm, table_hbm, out_hbm, idx_v, buf0, buf1, sem0, sem1)


def kernel(indices, table):
    idx = indices.reshape(_NW, _NCHUNK, _CHUNK).astype(jnp.int32)
    out = _gather(idx, table)
    return out.reshape(NUM_ROWS, NUM_FEATS, DIM)


# speedup vs baseline: 1.5811x; 1.0045x over previous
import functools

import jax
import jax.numpy as jnp
from jax import lax
from jax.experimental import pallas as pl
from jax.experimental.pallas import tpu as pltpu
from jax.experimental.pallas import tpu_sc as plsc

NUM_ROWS = 16384
NUM_FEATS = 26
DIM = 32

_NC = 2
_NS = 16
_NW = _NC * _NS

_B = NUM_ROWS * NUM_FEATS
_PER_W = _B // _NW
_CHUNK = 832
_NCHUNK = _PER_W // _CHUNK
_NBUF = 2


def _gather_body(idx_hbm, table_hbm, out_hbm, idx_v, buf0, buf1, sem0, sem1):
    wid = lax.axis_index("s") * _NC + lax.axis_index("c")
    base = wid * _PER_W
    pltpu.sync_copy(idx_hbm.at[pl.ds(base, _PER_W)], idx_v)
    bufs = (buf0, buf1)
    sems = (sem0, sem1)

    def start(j, b):
        pltpu.async_copy(
            table_hbm.at[idx_v.at[pl.ds(j * _CHUNK, _CHUNK)]], bufs[b], sems[b]
        )

    def finish(j, b):
        pltpu.make_async_copy(
            table_hbm.at[idx_v.at[pl.ds(j * _CHUNK, _CHUNK)]], bufs[b], sems[b]
        ).wait()
        pltpu.sync_copy(bufs[b], out_hbm.at[pl.ds(base + j * _CHUNK, _CHUNK)])

    for b in range(_NBUF):
        start(b, b)

    @pl.loop(0, _NCHUNK - _NBUF, step=_NBUF)
    def _(j0):
        for b in range(_NBUF):
            j = j0 + b
            finish(j, b)
            start(j + _NBUF, b)

    for b in range(_NBUF):
        finish(_NCHUNK - _NBUF + b, b)


@functools.partial(
    pl.kernel,
    out_type=jax.ShapeDtypeStruct((_B, DIM), jnp.float32),
    mesh=plsc.VectorSubcoreMesh(core_axis_name="c", subcore_axis_name="s"),
    scratch_types=[
        pltpu.VMEM((_PER_W,), jnp.int32),
        pltpu.VMEM((_CHUNK, DIM), jnp.float32),
        pltpu.VMEM((_CHUNK, DIM), jnp.float32),
        pltpu.SemaphoreType.DMA,
        pltpu.SemaphoreType.DMA,
    ],
    compiler_params=pltpu.CompilerParams(use_tc_tiling_on_sc=False),
)
def _gather(idx_hbm, table_hbm, out_hbm, idx_v, buf0, buf1, sem0, sem1):
    _gather_body(idx_hbm, table_hbm, out_hbm, idx_v, buf0, buf1, sem0, sem1)


def kernel(indices, table):
    out = _gather(indices.reshape(_B), table)
    return out.reshape(NUM_ROWS, NUM_FEATS, DIM)
